# baseline (speedup 1.0000x reference)
import jax
import jax.numpy as jnp
from jax.experimental import pallas as pl
from jax.experimental.pallas import tpu as pltpu


def _ngcn_kernel(adj_ref, x_ref, wcat_ref, bcat_ref, wfc_ref, bfc_ref,
                 o_ref, t_scr, u_scr, v_scr):
    p = pl.program_id(0)
    i = pl.program_id(1)
    bm = adj_ref.shape[0]
    nh = x_ref.shape[1]
    blk = pl.ds(i * bm, bm)

    @pl.when((p == 0) & (i == 0))
    def _stage_t():
        t_scr[...] = jnp.dot(x_ref[...], wcat_ref[...],
                             preferred_element_type=jnp.float32)

    @pl.when(p == 0)
    def _pass1():
        u_scr[blk, :] = jnp.dot(adj_ref[...], t_scr[:, nh:],
                                preferred_element_type=jnp.float32)

    @pl.when(p == 1)
    def _pass2():
        v_scr[blk, :] = jnp.dot(adj_ref[...], u_scr[...],
                                preferred_element_type=jnp.float32)

    @pl.when(p <= 1)
    def _fill_out():
        o_ref[...] = jnp.zeros_like(o_ref)

    @pl.when(p == 2)
    def _pass3_epilogue():
        w1 = jnp.dot(adj_ref[...], t_scr[:, :nh],
                     preferred_element_type=jnp.float32)
        w3 = jnp.dot(adj_ref[...], v_scr[:, nh:],
                     preferred_element_type=jnp.float32)
        h = jnp.concatenate([w1, v_scr[blk, :nh], w3], axis=1)
        h = jax.nn.relu(h + bcat_ref[...])
        logits = jnp.dot(h, wfc_ref[...], preferred_element_type=jnp.float32)
        o_ref[...] = jax.nn.sigmoid(logits + bfc_ref[...])


def _pick_bm(m, cap):
    for bm in (400, 200, 80, 40, 16, 8):
        if bm <= cap and m % bm == 0:
            return bm
    return m


def kernel(x, adj, W1, b1, W2, b2, W3, b3, Wfc, bfc):
    m, n = adj.shape
    nh = W1.shape[1]
    nl = Wfc.shape[1]
    kh = Wfc.shape[0]
    bm = _pick_bm(m, 200)

    wcat = jnp.concatenate([W1, W2, W3], axis=1)
    bcat = jnp.concatenate([b1, b2, b3])[None, :]

    return pl.pallas_call(
        _ngcn_kernel,
        grid=(3, m // bm),
        in_specs=[
            pl.BlockSpec((bm, n), lambda p, i: (i, 0)),
            pl.BlockSpec((n, nh), lambda p, i: (0, 0)),
            pl.BlockSpec((nh, kh), lambda p, i: (0, 0)),
            pl.BlockSpec((1, kh), lambda p, i: (0, 0)),
            pl.BlockSpec((kh, nl), lambda p, i: (0, 0)),
            pl.BlockSpec((1, nl), lambda p, i: (0, 0)),
        ],
        out_specs=pl.BlockSpec((bm, nl), lambda p, i: (i, 0)),
        out_shape=jax.ShapeDtypeStruct((m, nl), jnp.float32),
        scratch_shapes=[
            pltpu.VMEM((n, kh), jnp.float32),
            pltpu.VMEM((m, 2 * nh), jnp.float32),
            pltpu.VMEM((m, 2 * nh), jnp.float32),
        ],
        compiler_params=pltpu.CompilerParams(
            vmem_limit_bytes=63000000,
        ),
    )(adj, x, wcat, bcat, Wfc, bfc[None, :])


# speedup vs baseline: 1.0617x; 1.0617x over previous
import jax
import jax.numpy as jnp
from jax.experimental import pallas as pl
from jax.experimental.pallas import tpu as pltpu


def _ngcn_kernel(adj_ref, x_ref, wcat_ref, bcat_ref, wfc_ref, bfc_ref,
                 o_ref, a_scr, b_scr, acc_scr):
    p = pl.program_id(0)
    i = pl.program_id(1)
    bm = adj_ref.shape[0]
    nh = x_ref.shape[1]
    blk = pl.ds(i * bm, bm)

    @pl.when((p == 0) & (i == 0))
    def _stage_t23():
        a_scr[...] = jnp.dot(x_ref[...], wcat_ref[:, nh:],
                             preferred_element_type=jnp.float32)

    @pl.when(p == 0)
    def _pass1():
        b_scr[blk, :] = jnp.dot(adj_ref[...], a_scr[...],
                                preferred_element_type=jnp.float32)

    @pl.when(p == 1)
    def _pass2():
        v = jnp.dot(adj_ref[...], b_scr[...],
                    preferred_element_type=jnp.float32)
        h2 = jax.nn.relu(v[:, :nh] + bcat_ref[:, nh:2 * nh])
        acc_scr[blk, :] = jnp.dot(h2, wfc_ref[nh:2 * nh, :],
                                  preferred_element_type=jnp.float32)
        a_scr[blk, :nh] = v[:, nh:]

    @pl.when(p <= 1)
    def _fill_out():
        o_ref[...] = jnp.zeros_like(o_ref)

    @pl.when((p == 2) & (i == 0))
    def _stage_t1():
        a_scr[:, nh:] = jnp.dot(x_ref[...], wcat_ref[:, :nh],
                                preferred_element_type=jnp.float32)

    @pl.when(p == 2)
    def _pass3_epilogue():
        w1 = jnp.dot(adj_ref[...], a_scr[:, nh:],
                     preferred_element_type=jnp.float32)
        w3 = jnp.dot(adj_ref[...], a_scr[:, :nh],
                     preferred_element_type=jnp.float32)
        h1 = jax.nn.relu(w1 + bcat_ref[:, :nh])
        h3 = jax.nn.relu(w3 + bcat_ref[:, 2 * nh:])
        l1 = jnp.dot(h1, wfc_ref[:nh, :], preferred_element_type=jnp.float32)
        l3 = jnp.dot(h3, wfc_ref[2 * nh:, :],
                     preferred_element_type=jnp.float32)
        logits = (l1 + acc_scr[blk, :]) + l3 + bfc_ref[...]
        o_ref[...] = jax.nn.sigmoid(logits)


def _pick_bm(m, cap):
    for bm in (400, 200, 80, 40, 16, 8):
        if bm <= cap and m % bm == 0:
            return bm
    return m


def kernel(x, adj, W1, b1, W2, b2, W3, b3, Wfc, bfc):
    m, n = adj.shape
    nh = W1.shape[1]
    nl = Wfc.shape[1]
    kh = Wfc.shape[0]
    bm = _pick_bm(m, 400)

    wcat = jnp.concatenate([W1, W2, W3], axis=1)
    bcat = jnp.concatenate([b1, b2, b3])[None, :]

    return pl.pallas_call(
        _ngcn_kernel,
        grid=(3, m // bm),
        in_specs=[
            pl.BlockSpec((bm, n), lambda p, i: (i, 0)),
            pl.BlockSpec((n, nh), lambda p, i: (0, 0)),
            pl.BlockSpec((nh, kh), lambda p, i: (0, 0)),
            pl.BlockSpec((1, kh), lambda p, i: (0, 0)),
            pl.BlockSpec((kh, nl), lambda p, i: (0, 0)),
            pl.BlockSpec((1, nl), lambda p, i: (0, 0)),
        ],
        out_specs=pl.BlockSpec((bm, nl), lambda p, i: (i, 0)),
        out_shape=jax.ShapeDtypeStruct((m, nl), jnp.float32),
        scratch_shapes=[
            pltpu.VMEM((m, 2 * nh), jnp.float32),
            pltpu.VMEM((m, 2 * nh), jnp.float32),
            pltpu.VMEM((m, nl), jnp.float32),
        ],
        compiler_params=pltpu.CompilerParams(
            vmem_limit_bytes=66000000,
        ),
    )(adj, x, wcat, bcat, Wfc, bfc[None, :])


# speedup vs baseline: 1.0877x; 1.0244x over previous
import jax
import jax.numpy as jnp
from jax.experimental import pallas as pl
from jax.experimental.pallas import tpu as pltpu


def _ngcn_kernel(adj_ref, x_ref, wcat_ref, bcat_ref, wfc_ref, bfc_ref,
                 o_ref, a_scr, b_scr, acc_scr):
    p = pl.program_id(0)
    i = pl.program_id(1)
    bm = adj_ref.shape[0]
    nh = x_ref.shape[1]
    blk = pl.ds(i * bm, bm)

    @pl.when((p == 0) & (i == 0))
    def _stage_t23():
        a_scr[...] = jnp.dot(x_ref[...], wcat_ref[:, nh:],
                             preferred_element_type=jnp.float32)

    @pl.when(p == 0)
    def _pass1():
        b_scr[blk, :] = jnp.dot(adj_ref[...], a_scr[...],
                                preferred_element_type=jnp.float32)

    @pl.when(p == 1)
    def _pass2():
        v = jnp.dot(adj_ref[...], b_scr[...],
                    preferred_element_type=jnp.float32)
        h2 = jax.nn.relu(v[:, :nh] + bcat_ref[:, nh:2 * nh])
        acc_scr[blk, :] = jnp.dot(h2, wfc_ref[nh:2 * nh, :],
                                  preferred_element_type=jnp.float32)
        a_scr[blk, :nh] = v[:, nh:]

    @pl.when(p <= 1)
    def _fill_out():
        o_ref[...] = jnp.zeros_like(o_ref)

    @pl.when((p == 2) & (i == 0))
    def _stage_t1():
        a_scr[:, nh:] = jnp.dot(x_ref[...], wcat_ref[:, :nh],
                                preferred_element_type=jnp.float32)

    @pl.when(p == 2)
    def _pass3_epilogue():
        w31 = jnp.dot(adj_ref[...], a_scr[...],
                      preferred_element_type=jnp.float32)
        h1 = jax.nn.relu(w31[:, nh:] + bcat_ref[:, :nh])
        h3 = jax.nn.relu(w31[:, :nh] + bcat_ref[:, 2 * nh:])
        l1 = jnp.dot(h1, wfc_ref[:nh, :], preferred_element_type=jnp.float32)
        l3 = jnp.dot(h3, wfc_ref[2 * nh:, :],
                     preferred_element_type=jnp.float32)
        logits = (l1 + acc_scr[blk, :]) + l3 + bfc_ref[...]
        o_ref[...] = jax.nn.sigmoid(logits)


def _pick_bm(m, cap):
    for bm in (400, 200, 80, 40, 16, 8):
        if bm <= cap and m % bm == 0:
            return bm
    return m


def kernel(x, adj, W1, b1, W2, b2, W3, b3, Wfc, bfc):
    m, n = adj.shape
    nh = W1.shape[1]
    nl = Wfc.shape[1]
    kh = Wfc.shape[0]
    bm = _pick_bm(m, 400)

    wcat = jnp.concatenate([W1, W2, W3], axis=1)
    bcat = jnp.concatenate([b1, b2, b3])[None, :]

    return pl.pallas_call(
        _ngcn_kernel,
        grid=(3, m // bm),
        in_specs=[
            pl.BlockSpec((bm, n), lambda p, i: (i, 0)),
            pl.BlockSpec((n, nh), lambda p, i: (0, 0)),
            pl.BlockSpec((nh, kh), lambda p, i: (0, 0)),
            pl.BlockSpec((1, kh), lambda p, i: (0, 0)),
            pl.BlockSpec((kh, nl), lambda p, i: (0, 0)),
            pl.BlockSpec((1, nl), lambda p, i: (0, 0)),
        ],
        out_specs=pl.BlockSpec((bm, nl), lambda p, i: (i, 0)),
        out_shape=jax.ShapeDtypeStruct((m, nl), jnp.float32),
        scratch_shapes=[
            pltpu.VMEM((m, 2 * nh), jnp.float32),
            pltpu.VMEM((m, 2 * nh), jnp.float32),
            pltpu.VMEM((m, nl), jnp.float32),
        ],
        compiler_params=pltpu.CompilerParams(
            vmem_limit_bytes=66000000,
        ),
    )(adj, x, wcat, bcat, Wfc, bfc[None, :])
